# baseline (speedup 1.0000x reference)
import functools

import jax
import jax.numpy as jnp
from jax import lax
from jax.experimental import pallas as pl
from jax.experimental.pallas import tpu as pltpu
from jax.experimental.pallas import tpu_sc as plsc

N_NODES = 10000
N_EDGES = 320000
D = 128

NC = 2
NS = 16
NW = NC * NS
EPT = N_EDGES // NW
CHUNK = 80
G = 5
NGRP = EPT // (G * CHUNK)
NPAD = 10240
SLAB = NPAD // NS

f32 = jnp.float32

_mesh = plsc.VectorSubcoreMesh(core_axis_name="c", subcore_axis_name="s")


def _zero_slab(z_hbm, sh_ref, sid):
    pltpu.sync_copy(z_hbm.at[pl.ds(sid * SLAB, SLAB)],
                    sh_ref.at[pl.ds(sid * SLAB, SLAB)])


def _copy_out_slab(sh_ref, out_hbm, cid, sid):
    pltpu.sync_copy(sh_ref.at[pl.ds(sid * SLAB, SLAB)],
                    out_hbm.at[pl.ds(cid * NPAD + sid * SLAB, SLAB)])


_ROWS_PER_IT = 4


def _relu_add(a_v, b_v):
    @pl.loop(0, CHUNK, step=_ROWS_PER_IT)
    def _(i):
        for r in range(_ROWS_PER_IT):
            for q in range(0, D, 16):
                a_v[i + r, pl.ds(q, 16)] = jnp.maximum(
                    a_v[i + r, pl.ds(q, 16)] + b_v[i + r, pl.ds(q, 16)], 0.0)


def _plain_add(a_v, b_v):
    @pl.loop(0, CHUNK, step=_ROWS_PER_IT)
    def _(i):
        for r in range(_ROWS_PER_IT):
            for q in range(0, D, 16):
                a_v[i + r, pl.ds(q, 16)] = (a_v[i + r, pl.ds(q, 16)]
                                            + b_v[i + r, pl.ds(q, 16)])


@functools.partial(
    pl.kernel,
    out_type=jax.ShapeDtypeStruct((NC * NPAD, D), f32),
    mesh=_mesh,
    scratch_types=[
        pltpu.VMEM((G, CHUNK), jnp.int32),
        pltpu.VMEM((CHUNK, D), f32),
        pltpu.VMEM((CHUNK, D), f32),
        pltpu.VMEM_SHARED((NPAD, D), f32),
        pltpu.SemaphoreType.DMA,
        pltpu.SemaphoreType.DMA,
    ],
)
def _sc_scatter_x(x_hbm, dsti_hbm, z128_hbm,
                  acc_out, dsti_v, r0, r1, acc_sh, s0, s1):
    cid = lax.axis_index("c")
    sid = lax.axis_index("s")
    wid = cid * NS + sid
    _zero_slab(z128_hbm, acc_sh, sid)
    plsc.subcore_barrier()
    ebase = wid * EPT
    rbuf = (r0, r1)
    sem = (s0, s1)

    @pl.loop(0, NGRP)
    def _(g):
        pltpu.sync_copy(dsti_hbm.at[wid, g], dsti_v)
        gbase = ebase + g * G * CHUNK
        pltpu.async_copy(x_hbm.at[pl.ds(gbase, CHUNK)], rbuf[0], sem[0])
        for k in range(G):
            kb = k % 2
            if k + 1 < G:
                nb = (k + 1) % 2
                pltpu.async_copy(
                    x_hbm.at[pl.ds(gbase + (k + 1) * CHUNK, CHUNK)],
                    rbuf[nb], sem[nb])
            pltpu.make_async_copy(
                x_hbm.at[pl.ds(gbase + k * CHUNK, CHUNK)],
                rbuf[kb], sem[kb]).wait()
            pltpu.sync_copy(rbuf[kb], acc_sh.at[dsti_v.at[k]], add=True)

    plsc.subcore_barrier()
    _copy_out_slab(acc_sh, acc_out, cid, sid)


@functools.partial(
    pl.kernel,
    out_type=jax.ShapeDtypeStruct((NC * NPAD, D), f32),
    mesh=_mesh,
    scratch_types=[
        pltpu.VMEM((G, CHUNK), jnp.int32),
        pltpu.VMEM((CHUNK, D), f32),
        pltpu.VMEM_SHARED((NPAD, D), f32),
    ],
)
def _sc_deg(dsti_hbm, z128_hbm,
            deg_out, dsti_v, ones_v, deg_sh):
    cid = lax.axis_index("c")
    sid = lax.axis_index("s")
    wid = cid * NS + sid
    _zero_slab(z128_hbm, deg_sh, sid)

    @pl.loop(0, CHUNK)
    def _(i):
        @pl.loop(0, D, step=16)
        def _(q):
            ones_v[i, pl.ds(q, 16)] = jnp.ones((16,), f32)

    plsc.subcore_barrier()

    @pl.loop(0, NGRP)
    def _(g):
        pltpu.sync_copy(dsti_hbm.at[wid, g], dsti_v)
        for k in range(G):
            pltpu.sync_copy(ones_v, deg_sh.at[dsti_v.at[k]], add=True)

    plsc.subcore_barrier()
    _copy_out_slab(deg_sh, deg_out, cid, sid)


@functools.partial(
    pl.kernel,
    out_type=jax.ShapeDtypeStruct((NC * NPAD, D), f32),
    mesh=_mesh,
    scratch_types=[
        pltpu.VMEM((G, CHUNK), jnp.int32),
        pltpu.VMEM((G, CHUNK), jnp.int32),
        pltpu.VMEM((CHUNK, D), f32),
        pltpu.VMEM((CHUNK, D), f32),
        pltpu.VMEM((CHUNK, D), f32),
        pltpu.VMEM((CHUNK, D), f32),
        pltpu.VMEM_SHARED((NPAD, D), f32),
        pltpu.SemaphoreType.DMA,
        pltpu.SemaphoreType.DMA,
        pltpu.SemaphoreType.DMA,
        pltpu.SemaphoreType.DMA,
    ],
)
def _sc_msg_scatter(y_hbm, srci_hbm, dsti_hbm, z128_hbm,
                    acc_out, srci_v, dsti_v, a0, a1, b0, b1, acc_sh,
                    sa0, sa1, sb0, sb1):
    cid = lax.axis_index("c")
    sid = lax.axis_index("s")
    wid = cid * NS + sid
    _zero_slab(z128_hbm, acc_sh, sid)
    plsc.subcore_barrier()
    abuf = (a0, a1)
    bbuf = (b0, b1)
    sa = (sa0, sa1)
    sb = (sb0, sb1)

    @pl.loop(0, NGRP)
    def _(g):
        pltpu.sync_copy(srci_hbm.at[wid, g], srci_v)
        pltpu.sync_copy(dsti_hbm.at[wid, g], dsti_v)
        pltpu.async_copy(y_hbm.at[srci_v.at[0]], abuf[0], sa[0])
        pltpu.async_copy(y_hbm.at[dsti_v.at[0]], bbuf[0], sb[0])
        for k in range(G):
            kb = k % 2
            if k + 1 < G:
                nb = (k + 1) % 2
                pltpu.async_copy(y_hbm.at[srci_v.at[k + 1]], abuf[nb], sa[nb])
                pltpu.async_copy(y_hbm.at[dsti_v.at[k + 1]], bbuf[nb], sb[nb])
            pltpu.make_async_copy(
                y_hbm.at[srci_v.at[k]], abuf[kb], sa[kb]).wait()
            pltpu.make_async_copy(
                y_hbm.at[dsti_v.at[k]], bbuf[kb], sb[kb]).wait()
            _relu_add(abuf[kb], bbuf[kb])
            pltpu.sync_copy(abuf[kb], acc_sh.at[dsti_v.at[k]], add=True)

    plsc.subcore_barrier()
    _copy_out_slab(acc_sh, acc_out, cid, sid)


@functools.partial(
    pl.kernel,
    out_type=jax.ShapeDtypeStruct((NC * NPAD, D), f32),
    mesh=_mesh,
    scratch_types=[
        pltpu.VMEM((G, CHUNK), jnp.int32),
        pltpu.VMEM((G, CHUNK), jnp.int32),
        pltpu.VMEM((CHUNK, D), f32),
        pltpu.VMEM((CHUNK, D), f32),
        pltpu.VMEM_SHARED((NPAD, D), f32),
        pltpu.SemaphoreType.DMA,
        pltpu.SemaphoreType.DMA,
    ],
)
def _sc_fwd_scatter(nh_hbm, srci_hbm, dsti_hbm, z128_hbm,
                    h2_out, srci_v, dsti_v, a0, a1, h2_sh, sa0, sa1):
    cid = lax.axis_index("c")
    sid = lax.axis_index("s")
    wid = cid * NS + sid
    _zero_slab(z128_hbm, h2_sh, sid)
    plsc.subcore_barrier()
    abuf = (a0, a1)
    sa = (sa0, sa1)

    @pl.loop(0, NGRP)
    def _(g):
        pltpu.sync_copy(srci_hbm.at[wid, g], srci_v)
        pltpu.sync_copy(dsti_hbm.at[wid, g], dsti_v)
        pltpu.async_copy(nh_hbm.at[srci_v.at[0]], abuf[0], sa[0])
        for k in range(G):
            kb = k % 2
            if k + 1 < G:
                nb = (k + 1) % 2
                pltpu.async_copy(nh_hbm.at[srci_v.at[k + 1]], abuf[nb], sa[nb])
            pltpu.make_async_copy(
                nh_hbm.at[srci_v.at[k]], abuf[kb], sa[kb]).wait()
            pltpu.sync_copy(abuf[kb], h2_sh.at[dsti_v.at[k]], add=True)

    plsc.subcore_barrier()
    _copy_out_slab(h2_sh, h2_out, cid, sid)


@functools.partial(
    pl.kernel,
    out_type=jax.ShapeDtypeStruct((N_EDGES, D), f32),
    mesh=_mesh,
    scratch_types=[
        pltpu.VMEM((G, CHUNK), jnp.int32),
        pltpu.VMEM((G, CHUNK), jnp.int32),
        pltpu.VMEM((CHUNK, D), f32),
        pltpu.VMEM((CHUNK, D), f32),
        pltpu.VMEM((CHUNK, D), f32),
        pltpu.VMEM((CHUNK, D), f32),
        pltpu.SemaphoreType.DMA,
        pltpu.SemaphoreType.DMA,
        pltpu.SemaphoreType.DMA,
        pltpu.SemaphoreType.DMA,
    ],
)
def _sc_pair_sum(y_hbm, srci_hbm, dsti_hbm,
                 out_hbm, srci_v, dsti_v, a0, a1, b0, b1,
                 sa0, sa1, sb0, sb1):
    cid = lax.axis_index("c")
    sid = lax.axis_index("s")
    wid = cid * NS + sid
    ebase = wid * EPT
    abuf = (a0, a1)
    bbuf = (b0, b1)
    sa = (sa0, sa1)
    sb = (sb0, sb1)

    @pl.loop(0, NGRP)
    def _(g):
        pltpu.sync_copy(srci_hbm.at[wid, g], srci_v)
        pltpu.sync_copy(dsti_hbm.at[wid, g], dsti_v)
        pltpu.async_copy(y_hbm.at[srci_v.at[0]], abuf[0], sa[0])
        pltpu.async_copy(y_hbm.at[dsti_v.at[0]], bbuf[0], sb[0])
        for k in range(G):
            kb = k % 2
            if k + 1 < G:
                nb = (k + 1) % 2
                pltpu.async_copy(y_hbm.at[srci_v.at[k + 1]], abuf[nb], sa[nb])
                pltpu.async_copy(y_hbm.at[dsti_v.at[k + 1]], bbuf[nb], sb[nb])
            pltpu.make_async_copy(
                y_hbm.at[srci_v.at[k]], abuf[kb], sa[kb]).wait()
            pltpu.make_async_copy(
                y_hbm.at[dsti_v.at[k]], bbuf[kb], sb[kb]).wait()
            _plain_add(abuf[kb], bbuf[kb])
            pltpu.sync_copy(
                abuf[kb],
                out_hbm.at[pl.ds(ebase + (g * G + k) * CHUNK, CHUNK)])


_TC_BLK = 512
_TC_GRID = NPAD // _TC_BLK


def _tc_combine_body(a0_ref, a1_ref, d0_ref, d1_ref, nh_ref):
    deg = jnp.maximum(d0_ref[:, :1] + d1_ref[:, :1], 1.0)
    nh_ref[...] = (a0_ref[...] + a1_ref[...]) * (1.0 / deg)


def _tc_combine(acc_cat, deg_cat):
    return pl.pallas_call(
        _tc_combine_body,
        grid=(_TC_GRID,),
        in_specs=[
            pl.BlockSpec((_TC_BLK, D), lambda i: (i, 0)),
            pl.BlockSpec((_TC_BLK, D), lambda i: (i + _TC_GRID, 0)),
            pl.BlockSpec((_TC_BLK, D), lambda i: (i, 0)),
            pl.BlockSpec((_TC_BLK, D), lambda i: (i + _TC_GRID, 0)),
        ],
        out_specs=pl.BlockSpec((_TC_BLK, D), lambda i: (i, 0)),
        out_shape=jax.ShapeDtypeStruct((NPAD, D), f32),
    )(acc_cat, acc_cat, deg_cat, deg_cat)


def _tc_linear_body(h2_ref, w_ref, b_ref, y_ref):
    h = 0.5 * (h2_ref[:NPAD, :] + h2_ref[NPAD:, :])
    y_ref[...] = lax.dot_general(
        h, w_ref[...], (((1,), (1,)), ((), ())),
        precision=lax.Precision.HIGHEST,
        preferred_element_type=f32) + 0.5 * b_ref[...]


def _tc_linear(h2_cat, W, b):
    return pl.pallas_call(
        _tc_linear_body,
        out_shape=jax.ShapeDtypeStruct((NPAD, D), f32),
    )(h2_cat, W, b.reshape(1, D))


def kernel(x, edge_index, W1, b1, W2, b2, W3, b3, W4, b4, W5, b5):
    ei = edge_index.astype(jnp.int32)
    srci = ei[0].reshape(NW, NGRP, G, CHUNK)
    dsti = ei[1].reshape(NW, NGRP, G, CHUNK)
    z128 = jnp.zeros((NPAD, D), f32)

    acc = _sc_scatter_x(x, dsti, z128)
    deg = _sc_deg(dsti, z128)
    y = None
    for (W, b) in ((W1, b1), (W2, b2), (W3, b3), (W4, b4), (W5, b5)):
        if y is not None:
            acc = _sc_msg_scatter(y, srci, dsti, z128)
        nh = _tc_combine(acc, deg)
        h2 = _sc_fwd_scatter(nh, srci, dsti, z128)
        y = _tc_linear(h2, W, b)
    return _sc_pair_sum(y, srci, dsti)


# speedup vs baseline: 1.0251x; 1.0251x over previous
import functools

import jax
import jax.numpy as jnp
from jax import lax
from jax.experimental import pallas as pl
from jax.experimental.pallas import tpu as pltpu
from jax.experimental.pallas import tpu_sc as plsc

N_NODES = 10000
N_EDGES = 320000
D = 128

NC = 2
NS = 16
NW = NC * NS
EPT = N_EDGES // NW
CHUNK = 80
G = 5
NGRP = EPT // (G * CHUNK)
NPAD = 10240
SLAB = NPAD // NS

f32 = jnp.float32

_mesh = plsc.VectorSubcoreMesh(core_axis_name="c", subcore_axis_name="s")


def _zero_slab(z_hbm, sh_ref, sid):
    pltpu.sync_copy(z_hbm.at[pl.ds(sid * SLAB, SLAB)],
                    sh_ref.at[pl.ds(sid * SLAB, SLAB)])


def _copy_out_slab(sh_ref, out_hbm, cid, sid):
    pltpu.sync_copy(sh_ref.at[pl.ds(sid * SLAB, SLAB)],
                    out_hbm.at[pl.ds(cid * NPAD + sid * SLAB, SLAB)])


_ROWS_PER_IT = 4


def _relu_add(a_v, b_v):
    @pl.loop(0, CHUNK, step=_ROWS_PER_IT)
    def _(i):
        for r in range(_ROWS_PER_IT):
            for q in range(0, D, 16):
                a_v[i + r, pl.ds(q, 16)] = jnp.maximum(
                    a_v[i + r, pl.ds(q, 16)] + b_v[i + r, pl.ds(q, 16)], 0.0)


def _plain_add(a_v, b_v):
    @pl.loop(0, CHUNK, step=_ROWS_PER_IT)
    def _(i):
        for r in range(_ROWS_PER_IT):
            for q in range(0, D, 16):
                a_v[i + r, pl.ds(q, 16)] = (a_v[i + r, pl.ds(q, 16)]
                                            + b_v[i + r, pl.ds(q, 16)])


@functools.partial(
    pl.kernel,
    out_type=jax.ShapeDtypeStruct((NC * NPAD, D), f32),
    mesh=_mesh,
    scratch_types=[
        pltpu.VMEM((G, CHUNK), jnp.int32),
        pltpu.VMEM((CHUNK, D), f32),
        pltpu.VMEM((CHUNK, D), f32),
        pltpu.VMEM_SHARED((NPAD, D), f32),
        pltpu.SemaphoreType.DMA,
        pltpu.SemaphoreType.DMA,
    ],
)
def _sc_scatter_x(x_hbm, dsti_hbm, z128_hbm,
                  acc_out, dsti_v, r0, r1, acc_sh, s0, s1):
    cid = lax.axis_index("c")
    sid = lax.axis_index("s")
    wid = cid * NS + sid
    _zero_slab(z128_hbm, acc_sh, sid)
    plsc.subcore_barrier()
    ebase = wid * EPT
    rbuf = (r0, r1)
    sem = (s0, s1)

    @pl.loop(0, NGRP)
    def _(g):
        pltpu.sync_copy(dsti_hbm.at[wid, g], dsti_v)
        gbase = ebase + g * G * CHUNK
        pltpu.async_copy(x_hbm.at[pl.ds(gbase, CHUNK)], rbuf[0], sem[0])
        for k in range(G):
            kb = k % 2
            if k + 1 < G:
                nb = (k + 1) % 2
                pltpu.async_copy(
                    x_hbm.at[pl.ds(gbase + (k + 1) * CHUNK, CHUNK)],
                    rbuf[nb], sem[nb])
            pltpu.make_async_copy(
                x_hbm.at[pl.ds(gbase + k * CHUNK, CHUNK)],
                rbuf[kb], sem[kb]).wait()
            pltpu.sync_copy(rbuf[kb], acc_sh.at[dsti_v.at[k]], add=True)

    plsc.subcore_barrier()
    _copy_out_slab(acc_sh, acc_out, cid, sid)


@functools.partial(
    pl.kernel,
    out_type=jax.ShapeDtypeStruct((NC * NPAD, D), f32),
    mesh=_mesh,
    scratch_types=[
        pltpu.VMEM((G, CHUNK), jnp.int32),
        pltpu.VMEM((CHUNK, D), f32),
        pltpu.VMEM_SHARED((NPAD, D), f32),
    ],
)
def _sc_deg(dsti_hbm, z128_hbm,
            deg_out, dsti_v, ones_v, deg_sh):
    cid = lax.axis_index("c")
    sid = lax.axis_index("s")
    wid = cid * NS + sid
    _zero_slab(z128_hbm, deg_sh, sid)

    @pl.loop(0, CHUNK)
    def _(i):
        @pl.loop(0, D, step=16)
        def _(q):
            ones_v[i, pl.ds(q, 16)] = jnp.ones((16,), f32)

    plsc.subcore_barrier()

    @pl.loop(0, NGRP)
    def _(g):
        pltpu.sync_copy(dsti_hbm.at[wid, g], dsti_v)
        for k in range(G):
            pltpu.sync_copy(ones_v, deg_sh.at[dsti_v.at[k]], add=True)

    plsc.subcore_barrier()
    _copy_out_slab(deg_sh, deg_out, cid, sid)


@functools.partial(
    pl.kernel,
    out_type=jax.ShapeDtypeStruct((NC * NPAD, D), f32),
    mesh=_mesh,
    scratch_types=[
        pltpu.VMEM((G, CHUNK), jnp.int32),
        pltpu.VMEM((G, CHUNK), jnp.int32),
        pltpu.VMEM((CHUNK, D), f32),
        pltpu.VMEM((CHUNK, D), f32),
        pltpu.VMEM((CHUNK, D), f32),
        pltpu.VMEM((CHUNK, D), f32),
        pltpu.VMEM_SHARED((NPAD, D), f32),
        pltpu.SemaphoreType.DMA,
        pltpu.SemaphoreType.DMA,
        pltpu.SemaphoreType.DMA,
        pltpu.SemaphoreType.DMA,
    ],
)
def _sc_msg_scatter(y_hbm, srci_hbm, dsti_hbm, z128_hbm,
                    acc_out, srci_v, dsti_v, a0, a1, b0, b1, acc_sh,
                    sa0, sa1, sb0, sb1):
    cid = lax.axis_index("c")
    sid = lax.axis_index("s")
    wid = cid * NS + sid
    _zero_slab(z128_hbm, acc_sh, sid)
    plsc.subcore_barrier()
    abuf = (a0, a1)
    bbuf = (b0, b1)
    sa = (sa0, sa1)
    sb = (sb0, sb1)

    @pl.loop(0, NGRP)
    def _(g):
        pltpu.sync_copy(srci_hbm.at[wid, g], srci_v)
        pltpu.sync_copy(dsti_hbm.at[wid, g], dsti_v)
        pltpu.async_copy(y_hbm.at[srci_v.at[0]], abuf[0], sa[0])
        pltpu.async_copy(y_hbm.at[dsti_v.at[0]], bbuf[0], sb[0])
        for k in range(G):
            kb = k % 2
            if k + 1 < G:
                nb = (k + 1) % 2
                pltpu.async_copy(y_hbm.at[srci_v.at[k + 1]], abuf[nb], sa[nb])
                pltpu.async_copy(y_hbm.at[dsti_v.at[k + 1]], bbuf[nb], sb[nb])
            pltpu.make_async_copy(
                y_hbm.at[srci_v.at[k]], abuf[kb], sa[kb]).wait()
            pltpu.make_async_copy(
                y_hbm.at[dsti_v.at[k]], bbuf[kb], sb[kb]).wait()
            _relu_add(abuf[kb], bbuf[kb])
            pltpu.sync_copy(abuf[kb], acc_sh.at[dsti_v.at[k]], add=True)

    plsc.subcore_barrier()
    _copy_out_slab(acc_sh, acc_out, cid, sid)


@functools.partial(
    pl.kernel,
    out_type=jax.ShapeDtypeStruct((NC * NPAD, D), f32),
    mesh=_mesh,
    scratch_types=[
        pltpu.VMEM((G, CHUNK), jnp.int32),
        pltpu.VMEM((G, CHUNK), jnp.int32),
        pltpu.VMEM((CHUNK, D), f32),
        pltpu.VMEM((CHUNK, D), f32),
        pltpu.VMEM_SHARED((NPAD, D), f32),
        pltpu.SemaphoreType.DMA,
        pltpu.SemaphoreType.DMA,
    ],
)
def _sc_fwd_scatter(nh_hbm, srci_hbm, dsti_hbm, z128_hbm,
                    h2_out, srci_v, dsti_v, a0, a1, h2_sh, sa0, sa1):
    cid = lax.axis_index("c")
    sid = lax.axis_index("s")
    wid = cid * NS + sid
    _zero_slab(z128_hbm, h2_sh, sid)
    plsc.subcore_barrier()
    abuf = (a0, a1)
    sa = (sa0, sa1)

    @pl.loop(0, NGRP)
    def _(g):
        pltpu.sync_copy(srci_hbm.at[wid, g], srci_v)
        pltpu.sync_copy(dsti_hbm.at[wid, g], dsti_v)
        pltpu.async_copy(nh_hbm.at[srci_v.at[0]], abuf[0], sa[0])
        for k in range(G):
            kb = k % 2
            if k + 1 < G:
                nb = (k + 1) % 2
                pltpu.async_copy(nh_hbm.at[srci_v.at[k + 1]], abuf[nb], sa[nb])
            pltpu.make_async_copy(
                nh_hbm.at[srci_v.at[k]], abuf[kb], sa[kb]).wait()
            pltpu.sync_copy(abuf[kb], h2_sh.at[dsti_v.at[k]], add=True)

    plsc.subcore_barrier()
    _copy_out_slab(h2_sh, h2_out, cid, sid)


@functools.partial(
    pl.kernel,
    out_type=jax.ShapeDtypeStruct((N_EDGES, D), f32),
    mesh=_mesh,
    scratch_types=[
        pltpu.VMEM((G, CHUNK), jnp.int32),
        pltpu.VMEM((G, CHUNK), jnp.int32),
        pltpu.VMEM((CHUNK, D), f32),
        pltpu.VMEM((CHUNK, D), f32),
        pltpu.VMEM((CHUNK, D), f32),
        pltpu.VMEM((CHUNK, D), f32),
        pltpu.VMEM_SHARED((NPAD, D), f32),
        pltpu.SemaphoreType.DMA,
        pltpu.SemaphoreType.DMA,
        pltpu.SemaphoreType.DMA,
        pltpu.SemaphoreType.DMA,
    ],
)
def _sc_pair_sum(y_hbm, srci_hbm, dsti_hbm,
                 out_hbm, srci_v, dsti_v, a0, a1, b0, b1, y_sh,
                 sa0, sa1, sb0, sb1):
    cid = lax.axis_index("c")
    sid = lax.axis_index("s")
    wid = cid * NS + sid
    ebase = wid * EPT
    abuf = (a0, a1)
    bbuf = (b0, b1)
    sa = (sa0, sa1)
    sb = (sb0, sb1)

    pltpu.sync_copy(y_hbm.at[pl.ds(sid * SLAB, SLAB)],
                    y_sh.at[pl.ds(sid * SLAB, SLAB)])
    plsc.subcore_barrier()

    @pl.loop(0, NGRP)
    def _(g):
        pltpu.sync_copy(srci_hbm.at[wid, g], srci_v)
        pltpu.sync_copy(dsti_hbm.at[wid, g], dsti_v)
        pltpu.async_copy(y_sh.at[srci_v.at[0]], abuf[0], sa[0])
        pltpu.async_copy(y_sh.at[dsti_v.at[0]], bbuf[0], sb[0])
        for k in range(G):
            kb = k % 2
            if k + 1 < G:
                nb = (k + 1) % 2
                pltpu.async_copy(y_sh.at[srci_v.at[k + 1]], abuf[nb], sa[nb])
                pltpu.async_copy(y_sh.at[dsti_v.at[k + 1]], bbuf[nb], sb[nb])
            pltpu.make_async_copy(
                y_sh.at[srci_v.at[k]], abuf[kb], sa[kb]).wait()
            pltpu.make_async_copy(
                y_sh.at[dsti_v.at[k]], bbuf[kb], sb[kb]).wait()
            _plain_add(abuf[kb], bbuf[kb])
            pltpu.sync_copy(
                abuf[kb],
                out_hbm.at[pl.ds(ebase + (g * G + k) * CHUNK, CHUNK)])


_TC_BLK = 512
_TC_GRID = NPAD // _TC_BLK


def _tc_combine_body(a0_ref, a1_ref, d0_ref, d1_ref, nh_ref):
    deg = jnp.maximum(d0_ref[:, :1] + d1_ref[:, :1], 1.0)
    nh_ref[...] = (a0_ref[...] + a1_ref[...]) * (1.0 / deg)


def _tc_combine(acc_cat, deg_cat):
    return pl.pallas_call(
        _tc_combine_body,
        grid=(_TC_GRID,),
        in_specs=[
            pl.BlockSpec((_TC_BLK, D), lambda i: (i, 0)),
            pl.BlockSpec((_TC_BLK, D), lambda i: (i + _TC_GRID, 0)),
            pl.BlockSpec((_TC_BLK, D), lambda i: (i, 0)),
            pl.BlockSpec((_TC_BLK, D), lambda i: (i + _TC_GRID, 0)),
        ],
        out_specs=pl.BlockSpec((_TC_BLK, D), lambda i: (i, 0)),
        out_shape=jax.ShapeDtypeStruct((NPAD, D), f32),
    )(acc_cat, acc_cat, deg_cat, deg_cat)


def _tc_linear_body(h2_ref, w_ref, b_ref, y_ref):
    h = 0.5 * (h2_ref[:NPAD, :] + h2_ref[NPAD:, :])
    y_ref[...] = lax.dot_general(
        h, w_ref[...], (((1,), (1,)), ((), ())),
        precision=lax.Precision.HIGHEST,
        preferred_element_type=f32) + 0.5 * b_ref[...]


def _tc_linear(h2_cat, W, b):
    return pl.pallas_call(
        _tc_linear_body,
        out_shape=jax.ShapeDtypeStruct((NPAD, D), f32),
    )(h2_cat, W, b.reshape(1, D))


def kernel(x, edge_index, W1, b1, W2, b2, W3, b3, W4, b4, W5, b5):
    ei = edge_index.astype(jnp.int32)
    srci = ei[0].reshape(NW, NGRP, G, CHUNK)
    dsti = ei[1].reshape(NW, NGRP, G, CHUNK)
    z128 = jnp.zeros((NPAD, D), f32)

    acc = _sc_scatter_x(x, dsti, z128)
    deg = _sc_deg(dsti, z128)
    y = None
    for (W, b) in ((W1, b1), (W2, b2), (W3, b3), (W4, b4), (W5, b5)):
        if y is not None:
            acc = _sc_msg_scatter(y, srci, dsti, z128)
        nh = _tc_combine(acc, deg)
        h2 = _sc_fwd_scatter(nh, srci, dsti, z128)
        y = _tc_linear(h2, W, b)
    return _sc_pair_sum(y, srci, dsti)
